# baseline (speedup 1.0000x reference)
import functools

import jax
import jax.numpy as jnp
from jax import lax
from jax.experimental import pallas as pl
from jax.experimental.pallas import tpu as pltpu
from jax.experimental.pallas import tpu_sc as plsc

N = 10000
F_IN = 128
H = 128
C = 64

_NC = 2
_NS = 16
_NW = _NC * _NS
_L = 16


def _npad(n):
    blk = _NS * 128
    return -(-n // blk) * blk


def _edge_batch(ept, feat, npad, nbuf):
    for b in range(128, 0, -8):
        if ept % b or (ept // b) % nbuf:
            continue
        if (npad * feat + _NS * 2 * ept <= 2_080_000
                and (2 * nbuf + 1) * b * feat + 2 * ept <= 131_000):
            return b
    raise ValueError((ept, feat))


def _make_propagate(n_nodes, n_edges, feat):
    NBUF = 5
    npad = _npad(n_nodes)
    ept = n_edges // _NW
    assert ept * _NW == n_edges
    B = _edge_batch(ept, feat, npad, NBUF)
    nb = ept // B
    nbr = nb // NBUF
    assert nb == nbr * NBUF
    rpt = npad // _NS
    nzr = rpt // B
    assert nzr * B == rpt

    mesh = plsc.VectorSubcoreMesh(core_axis_name="c", subcore_axis_name="s")

    @functools.partial(
        pl.kernel,
        mesh=mesh,
        out_type=jax.ShapeDtypeStruct((_NC, npad, feat), jnp.float32),
        compiler_params=pltpu.CompilerParams(use_tc_tiling_on_sc=False),
        scratch_types=[
            pltpu.VMEM((nb, B), jnp.int32),
            pltpu.VMEM((nb, B), jnp.int32),
            pltpu.VMEM_SHARED((npad, feat), jnp.float32),
        ] + [pltpu.SemaphoreType.DMA for _ in range(NBUF + 1)],
    )
    def prop(tab_hbm, src3_hbm, dst3_hbm, out_hbm, sidx2, didx2, acc, *sems):
        gsem = sems[:NBUF]
        ssem = sems[NBUF]
        pl.run_scoped(
            lambda *rows: _prop_body(tab_hbm, src3_hbm, dst3_hbm, out_hbm,
                                     sidx2, didx2, acc, gsem, ssem, rows),
            *[pltpu.VMEM((B, feat), jnp.float32) for _ in range(NBUF)])

    def _prop_body(tab_hbm, src3_hbm, dst3_hbm, out_hbm, sidx2, didx2,
                   acc, gsem, ssem, rows):
        cid = lax.axis_index("c")
        sid = lax.axis_index("s")
        wid = cid * _NS + sid
        tb = sid * rpt
        r0 = rows[0]

        def zrow(i, _):
            def zlane(j, _):
                r0[i, pl.ds(j * _L, _L)] = jnp.zeros((_L,), jnp.float32)
                return 0
            return lax.fori_loop(0, feat // _L, zlane, 0)
        lax.fori_loop(0, B, zrow, 0)

        zd = [pltpu.async_copy(r0, acc.at[pl.ds(tb + r * B, B), :], ssem)
              for r in range(nzr)]
        for d in zd:
            d.wait()

        pltpu.sync_copy(src3_hbm.at[wid], sidx2)
        pltpu.sync_copy(dst3_hbm.at[wid], didx2)
        plsc.subcore_barrier()

        def round_(j, _):
            t0 = NBUF * j
            gd = [pltpu.async_copy(tab_hbm.at[sidx2.at[t0 + k]], rows[k], gsem[k])
                  for k in range(NBUF)]
            sd = []
            for k in range(NBUF):
                gd[k].wait()
                sd.append(pltpu.async_copy(
                    rows[k], acc.at[didx2.at[t0 + k]], ssem, add=True))
            for d in sd:
                d.wait()
            return 0
        lax.fori_loop(0, nbr, round_, 0)

        plsc.subcore_barrier()

        rd = [None] * nzr
        wd = [None] * nzr
        for r in range(nzr):
            k = r % NBUF
            if r >= NBUF:
                wd[r - NBUF].wait()
            rd[r] = pltpu.async_copy(acc.at[pl.ds(tb + r * B, B), :], rows[k],
                                     gsem[k])
            rd[r].wait()
            wd[r] = pltpu.async_copy(
                rows[k], out_hbm.at[cid, pl.ds(tb + r * B, B), :], gsem[k])
        for r in range(max(nzr - NBUF, 0), nzr):
            wd[r].wait()

    return prop


def _split_batch(ept, fh, npad):
    for b in range(128, 0, -8):
        if ept % b or (ept // b) % 5:
            continue
        if (npad * fh + _NS * 2 * ept <= 2_080_000
                and 11 * b * fh + 2 * ept + 2 * b * _L <= 120_000):
            return b
    raise ValueError((ept, fh))


def _make_propagate_split(n_nodes, n_edges, fh):
    NBUF = 5
    npad = _npad(n_nodes)
    ept = n_edges // _NS
    assert ept * _NS == n_edges
    B = _split_batch(ept, fh, npad)
    nb = ept // B
    nbr = nb // NBUF
    rpt = npad // _NS
    nzr = rpt // B
    assert nzr * B == rpt

    mesh = plsc.VectorSubcoreMesh(core_axis_name="c", subcore_axis_name="s")

    @functools.partial(
        pl.kernel,
        mesh=mesh,
        out_type=jax.ShapeDtypeStruct((_NC, npad, fh), jnp.float32),
        compiler_params=pltpu.CompilerParams(use_tc_tiling_on_sc=False),
        scratch_types=[
            pltpu.VMEM((nb, B), jnp.int32),
            pltpu.VMEM((nb, B), jnp.int32),
            pltpu.VMEM((B, _L), jnp.float32),
            pltpu.VMEM((B, fh), jnp.float32),
            pltpu.VMEM_SHARED((npad, fh), jnp.float32),
        ] + [pltpu.SemaphoreType.DMA for _ in range(NBUF + 1)],
    )
    def prop(tab_hbm, src3_hbm, dst3_hbm, hw_hbm, dis_hbm, out_hbm,
             sidx2, didx2, disbuf, hwbuf, acc, *sems):
        gsem = sems[:NBUF]
        ssem = sems[NBUF]
        pl.run_scoped(
            lambda *rows: _body(tab_hbm, src3_hbm, dst3_hbm, hw_hbm, dis_hbm,
                                out_hbm, sidx2, didx2, disbuf, hwbuf, acc,
                                gsem, ssem, rows),
            *[pltpu.VMEM((B, fh), jnp.float32) for _ in range(NBUF)])

    def _body(tab_hbm, src3_hbm, dst3_hbm, hw_hbm, dis_hbm, out_hbm,
              sidx2, didx2, disbuf, hwbuf, acc, gsem, ssem, rows):
        cid = lax.axis_index("c")
        sid = lax.axis_index("s")
        tb = sid * rpt
        r0 = rows[0]

        def zrow(i, _):
            def zlane(j, _):
                r0[i, pl.ds(j * _L, _L)] = jnp.zeros((_L,), jnp.float32)
                return 0
            return lax.fori_loop(0, fh // _L, zlane, 0)
        lax.fori_loop(0, B, zrow, 0)

        zd = [pltpu.async_copy(r0, acc.at[pl.ds(tb + r * B, B), :], ssem)
              for r in range(nzr)]
        for d in zd:
            d.wait()

        pltpu.sync_copy(src3_hbm.at[sid], sidx2)
        pltpu.sync_copy(dst3_hbm.at[sid], didx2)
        plsc.subcore_barrier()

        def round_(j, _):
            t0 = NBUF * j
            gd = [pltpu.async_copy(tab_hbm.at[cid].at[sidx2.at[t0 + k]],
                                   rows[k], gsem[k])
                  for k in range(NBUF)]
            sd = []
            for k in range(NBUF):
                gd[k].wait()
                sd.append(pltpu.async_copy(
                    rows[k], acc.at[didx2.at[t0 + k]], ssem, add=True))
            for d in sd:
                d.wait()
            return 0
        lax.fori_loop(0, nbr, round_, 0)

        plsc.subcore_barrier()

        wd = [None] * nzr
        for r in range(nzr):
            k = r % NBUF
            if r >= NBUF:
                wd[r - NBUF].wait()
            rb = tb + r * B
            rd = pltpu.async_copy(acc.at[pl.ds(rb, B), :], rows[k], gsem[k])
            pltpu.sync_copy(dis_hbm.at[pl.ds(rb, B), :], disbuf)
            pltpu.sync_copy(hw_hbm.at[cid, pl.ds(rb, B), :], hwbuf)
            rd.wait()
            rk = rows[k]

            def frow(i, _):
                dis_i = disbuf[i, :]

                def flane(j, _):
                    sl = pl.ds(j * _L, _L)
                    rk[i, sl] = hwbuf[i, sl] - dis_i * rk[i, sl]
                    return 0
                return lax.fori_loop(0, fh // _L, flane, 0)
            lax.fori_loop(0, B, frow, 0)
            wd[r] = pltpu.async_copy(
                rk, out_hbm.at[cid, pl.ds(rb, B), :], gsem[k])
        for r in range(max(nzr - NBUF, 0), nzr):
            wd[r].wait()

    return prop


def _make_degree(n_nodes, n_edges):
    feat = _L
    npad = _npad(n_nodes)
    ept = n_edges // _NW
    assert ept * _NW == n_edges
    B = _edge_batch(ept, feat, npad, 5)
    nb = ept // B
    nbr = nb // 5
    assert nb == nbr * 5 and nbr >= 2
    rpt = npad // _NS
    zr = rpt // 128

    mesh = plsc.VectorSubcoreMesh(core_axis_name="c", subcore_axis_name="s")

    @functools.partial(
        pl.kernel,
        mesh=mesh,
        out_type=jax.ShapeDtypeStruct((_NC, npad, feat), jnp.float32),
        compiler_params=pltpu.CompilerParams(use_tc_tiling_on_sc=False),
        scratch_types=[
            pltpu.VMEM((nb, B), jnp.int32),
            pltpu.VMEM((B, feat), jnp.float32),
            pltpu.VMEM((128, feat), jnp.float32),
            pltpu.VMEM_SHARED((npad, feat), jnp.float32),
            pltpu.SemaphoreType.DMA,
        ],
    )
    def degree(src3_hbm, out_hbm, sidx2, ones, zbuf, acc, ssem):
        cid = lax.axis_index("c")
        sid = lax.axis_index("s")
        wid = cid * _NS + sid
        tb = sid * rpt

        def orow(i, _):
            ones[i, :] = jnp.ones((_L,), jnp.float32)
            return 0
        lax.fori_loop(0, B, orow, 0)

        def zrow(i, _):
            zbuf[i, :] = jnp.zeros((_L,), jnp.float32)
            return 0
        lax.fori_loop(0, 128, zrow, 0)

        def zacc(rnd, _):
            pltpu.sync_copy(zbuf, acc.at[pl.ds(tb + rnd * 128, 128), :])
            return 0
        lax.fori_loop(0, zr, zacc, 0)

        pltpu.sync_copy(src3_hbm.at[wid], sidx2)
        plsc.subcore_barrier()

        def round_(j, _):
            sd = [pltpu.async_copy(ones, acc.at[sidx2.at[5 * j + k]], ssem,
                                   add=True)
                  for k in range(5)]
            for d in sd:
                d.wait()
            return 0
        lax.fori_loop(0, nbr, round_, 0)

        plsc.subcore_barrier()

        def readback(rnd, _):
            pltpu.sync_copy(acc.at[pl.ds(tb + rnd * 128, 128), :], zbuf)
            pltpu.sync_copy(zbuf, out_hbm.at[cid, pl.ds(tb + rnd * 128, 128), :])
            return 0
        lax.fori_loop(0, zr, readback, 0)

    return degree


def _dis_of(degm_ref):
    deg = degm_ref[0, :, 0:1] + degm_ref[1, :, 0:1]
    return jnp.where(deg > 0, lax.rsqrt(deg), 0.0)


def _tc1_body(x_ref, w0_ref, w1_ref, b_ref, degm_ref, xw0b_ref, z1_ref):
    xb = x_ref[...]
    dis = _dis_of(degm_ref)
    xw0b_ref[...] = (
        jnp.dot(xb, w0_ref[...], preferred_element_type=jnp.float32) + b_ref[...]
    )
    z1_ref[...] = dis * jnp.dot(xb, w1_ref[...], preferred_element_type=jnp.float32)


def _tc2_body(xw0b_ref, p1_ref, degm_ref, w0_ref, w1_ref, b_ref,
              hw0b_ref, z2_ref, dis16_ref):
    dis = _dis_of(degm_ref)
    h = jnp.maximum(xw0b_ref[...] - dis * (p1_ref[0] + p1_ref[1]), 0.0)
    hw0b_ref[...] = (
        jnp.dot(h, w0_ref[...], preferred_element_type=jnp.float32) + b_ref[...]
    )
    z2_ref[...] = dis * jnp.dot(h, w1_ref[...], preferred_element_type=jnp.float32)
    dis16_ref[...] = jnp.broadcast_to(dis, (dis.shape[0], _L))


_RB = 1000


def _full(shape):
    return pl.BlockSpec(shape, lambda i: (0,) * len(shape))


def _rows(feat):
    return pl.BlockSpec((_RB, feat), lambda i: (i, 0))


def _deg_spec():
    return pl.BlockSpec((_NC, _RB, _L), lambda i: (0, i, 0))


def _part_spec(feat):
    return pl.BlockSpec((_NC, _RB, feat), lambda i: (0, i, 0))


def kernel(x, adj, W0_1, W1_1, b1, W0_2, W1_2, b2):
    n, f_in = x.shape
    e = adj.shape[1]
    h = W0_1.shape[1]
    c = W0_2.shape[1]
    grid = (n // _RB,)

    ept = e // _NW
    npad = _npad(n)
    Bd = _edge_batch(ept, _L, npad, 5)
    B1 = _edge_batch(ept, h, npad, 5)

    def _r3(v, b):
        return v.reshape(_NW, ept // b, b)

    fh = c // _NC
    degree = _make_degree(n, e)
    prop1 = _make_propagate(n, e, h)
    prop2 = _make_propagate_split(n, e, fh)

    degm = degree(_r3(adj[0], Bd))

    tc1 = pl.pallas_call(
        _tc1_body,
        grid=grid,
        in_specs=[
            _rows(f_in), _full((f_in, h)), _full((f_in, h)), _full((1, h)),
            _deg_spec(),
        ],
        out_specs=[_rows(h), _rows(h)],
        out_shape=[
            jax.ShapeDtypeStruct((n, h), jnp.float32),
            jax.ShapeDtypeStruct((n, h), jnp.float32),
        ],
    )
    xw0b, z1 = tc1(x, W0_1, W1_1, b1.reshape(1, h), degm)

    p1 = prop1(z1, _r3(adj[0], B1), _r3(adj[1], B1))

    tc2 = pl.pallas_call(
        _tc2_body,
        grid=grid,
        in_specs=[
            _rows(h), _part_spec(h), _deg_spec(),
            _full((h, c)), _full((h, c)), _full((1, c)),
        ],
        out_specs=[_rows(c), _rows(c), _rows(_L)],
        out_shape=[
            jax.ShapeDtypeStruct((n, c), jnp.float32),
            jax.ShapeDtypeStruct((n, c), jnp.float32),
            jax.ShapeDtypeStruct((n, _L), jnp.float32),
        ],
    )
    hw0b, z2, dis16 = tc2(xw0b, p1, degm, W0_2, W1_2, b2.reshape(1, c))

    pad = ((0, npad - n), (0, 0))
    z2s = jnp.stack([z2[:, :fh], z2[:, fh:]])
    hw_s = jnp.pad(jnp.stack([hw0b[:, :fh], hw0b[:, fh:]]),
                   ((0, 0),) + pad)
    dis_p = jnp.pad(dis16, pad)
    ept2 = e // _NS
    b2s = _split_batch(ept2, fh, npad)
    src3s = adj[0].reshape(_NS, ept2 // b2s, b2s)
    dst3s = adj[1].reshape(_NS, ept2 // b2s, b2s)
    out2 = prop2(z2s, src3s, dst3s, hw_s, dis_p)
    return jnp.concatenate([out2[0, :n, :], out2[1, :n, :]], axis=1)


# speedup vs baseline: 1.1601x; 1.1601x over previous
import functools

import jax
import jax.numpy as jnp
from jax import lax
from jax.experimental import pallas as pl
from jax.experimental.pallas import tpu as pltpu
from jax.experimental.pallas import tpu_sc as plsc

N = 10000
F_IN = 128
H = 128
C = 64

_NC = 2
_NS = 16
_NW = _NC * _NS
_L = 16


def _npad(n):
    blk = _NS * 128
    return -(-n // blk) * blk


def _edge_batch(ept, feat, npad, nbuf):
    for b in range(128, 0, -8):
        if ept % b or (ept // b) % nbuf:
            continue
        if (npad * feat + _NS * 2 * ept <= 2_080_000
                and (2 * nbuf + 1) * b * feat + 2 * ept <= 131_000):
            return b
    raise ValueError((ept, feat))


def _make_propagate(n_nodes, n_edges, feat):
    NBUF = 5
    npad = _npad(n_nodes)
    ept = n_edges // _NW
    assert ept * _NW == n_edges
    B = _edge_batch(ept, feat, npad, NBUF)
    nb = ept // B
    nbr = nb // NBUF
    assert nb == nbr * NBUF
    rpt = npad // _NS
    nzr = rpt // B
    assert nzr * B == rpt

    mesh = plsc.VectorSubcoreMesh(core_axis_name="c", subcore_axis_name="s")

    @functools.partial(
        pl.kernel,
        mesh=mesh,
        out_type=jax.ShapeDtypeStruct((_NC, npad, feat), jnp.float32),
        compiler_params=pltpu.CompilerParams(use_tc_tiling_on_sc=False),
        scratch_types=[
            pltpu.VMEM((nb, B), jnp.int32),
            pltpu.VMEM((nb, B), jnp.int32),
            pltpu.VMEM_SHARED((npad, feat), jnp.float32),
        ] + [pltpu.SemaphoreType.DMA for _ in range(NBUF + 1)],
    )
    def prop(tab_hbm, src3_hbm, dst3_hbm, out_hbm, sidx2, didx2, acc, *sems):
        gsem = sems[:NBUF]
        ssem = sems[NBUF]
        pl.run_scoped(
            lambda *rows: _prop_body(tab_hbm, src3_hbm, dst3_hbm, out_hbm,
                                     sidx2, didx2, acc, gsem, ssem, rows),
            *[pltpu.VMEM((B, feat), jnp.float32) for _ in range(NBUF)])

    def _prop_body(tab_hbm, src3_hbm, dst3_hbm, out_hbm, sidx2, didx2,
                   acc, gsem, ssem, rows):
        cid = lax.axis_index("c")
        sid = lax.axis_index("s")
        wid = cid * _NS + sid
        tb = sid * rpt
        r0 = rows[0]

        def zrow(i, _):
            def zlane(j, _):
                r0[i, pl.ds(j * _L, _L)] = jnp.zeros((_L,), jnp.float32)
                return 0
            return lax.fori_loop(0, feat // _L, zlane, 0)
        lax.fori_loop(0, B, zrow, 0)

        zd = [pltpu.async_copy(r0, acc.at[pl.ds(tb + r * B, B), :], ssem)
              for r in range(nzr)]
        for d in zd:
            d.wait()

        pltpu.sync_copy(src3_hbm.at[wid], sidx2)
        pltpu.sync_copy(dst3_hbm.at[wid], didx2)
        plsc.subcore_barrier()

        def round_(j, _):
            t0 = NBUF * j
            gd = [pltpu.async_copy(tab_hbm.at[sidx2.at[t0 + k]], rows[k], gsem[k])
                  for k in range(NBUF)]
            sd = []
            for k in range(NBUF):
                gd[k].wait()
                sd.append(pltpu.async_copy(
                    rows[k], acc.at[didx2.at[t0 + k]], ssem, add=True))
            for d in sd:
                d.wait()
            return 0
        lax.fori_loop(0, nbr, round_, 0)

        plsc.subcore_barrier()

        rd = [None] * nzr
        wd = [None] * nzr
        for r in range(nzr):
            k = r % NBUF
            if r >= NBUF:
                wd[r - NBUF].wait()
            rd[r] = pltpu.async_copy(acc.at[pl.ds(tb + r * B, B), :], rows[k],
                                     gsem[k])
            rd[r].wait()
            wd[r] = pltpu.async_copy(
                rows[k], out_hbm.at[cid, pl.ds(tb + r * B, B), :], gsem[k])
        for r in range(max(nzr - NBUF, 0), nzr):
            wd[r].wait()

    return prop


def _make_degree(n_nodes, n_edges):
    feat = _L
    npad = _npad(n_nodes)
    ept = n_edges // _NW
    assert ept * _NW == n_edges
    B = _edge_batch(ept, feat, npad, 5)
    nb = ept // B
    nbr = nb // 5
    assert nb == nbr * 5 and nbr >= 2
    rpt = npad // _NS
    zr = rpt // 128

    mesh = plsc.VectorSubcoreMesh(core_axis_name="c", subcore_axis_name="s")

    @functools.partial(
        pl.kernel,
        mesh=mesh,
        out_type=jax.ShapeDtypeStruct((_NC, npad, feat), jnp.float32),
        compiler_params=pltpu.CompilerParams(use_tc_tiling_on_sc=False),
        scratch_types=[
            pltpu.VMEM((nb, B), jnp.int32),
            pltpu.VMEM((B, feat), jnp.float32),
            pltpu.VMEM((128, feat), jnp.float32),
            pltpu.VMEM_SHARED((npad, feat), jnp.float32),
            pltpu.SemaphoreType.DMA,
        ],
    )
    def degree(src3_hbm, out_hbm, sidx2, ones, zbuf, acc, ssem):
        cid = lax.axis_index("c")
        sid = lax.axis_index("s")
        wid = cid * _NS + sid
        tb = sid * rpt

        def orow(i, _):
            ones[i, :] = jnp.ones((_L,), jnp.float32)
            return 0
        lax.fori_loop(0, B, orow, 0)

        def zrow(i, _):
            zbuf[i, :] = jnp.zeros((_L,), jnp.float32)
            return 0
        lax.fori_loop(0, 128, zrow, 0)

        def zacc(rnd, _):
            pltpu.sync_copy(zbuf, acc.at[pl.ds(tb + rnd * 128, 128), :])
            return 0
        lax.fori_loop(0, zr, zacc, 0)

        pltpu.sync_copy(src3_hbm.at[wid], sidx2)
        plsc.subcore_barrier()

        def round_(j, _):
            sd = [pltpu.async_copy(ones, acc.at[sidx2.at[5 * j + k]], ssem,
                                   add=True)
                  for k in range(5)]
            for d in sd:
                d.wait()
            return 0
        lax.fori_loop(0, nbr, round_, 0)

        plsc.subcore_barrier()

        def readback(rnd, _):
            pltpu.sync_copy(acc.at[pl.ds(tb + rnd * 128, 128), :], zbuf)
            pltpu.sync_copy(zbuf, out_hbm.at[cid, pl.ds(tb + rnd * 128, 128), :])
            return 0
        lax.fori_loop(0, zr, readback, 0)

    return degree


def _dis_of(degm_ref):
    deg = degm_ref[0, :, 0:1] + degm_ref[1, :, 0:1]
    return jnp.where(deg > 0, lax.rsqrt(deg), 0.0)


def _tc1_body(x_ref, w0_ref, w1_ref, b_ref, degm_ref, xw0b_ref, z1_ref):
    xb = x_ref[...]
    dis = _dis_of(degm_ref)
    xw0b_ref[...] = (
        jnp.dot(xb, w0_ref[...], preferred_element_type=jnp.float32) + b_ref[...]
    )
    z1_ref[...] = dis * jnp.dot(xb, w1_ref[...], preferred_element_type=jnp.float32)


def _tc2_body(xw0b_ref, p1_ref, degm_ref, w0_ref, w1_ref, b_ref, hw0b_ref, z2_ref):
    dis = _dis_of(degm_ref)
    h = jnp.maximum(xw0b_ref[...] - dis * (p1_ref[0] + p1_ref[1]), 0.0)
    hw0b_ref[...] = (
        jnp.dot(h, w0_ref[...], preferred_element_type=jnp.float32) + b_ref[...]
    )
    z2_ref[...] = dis * jnp.dot(h, w1_ref[...], preferred_element_type=jnp.float32)


def _tc3_body(hw0b_ref, p2_ref, degm_ref, out_ref):
    dis = _dis_of(degm_ref)
    out_ref[...] = hw0b_ref[...] - dis * (p2_ref[0] + p2_ref[1])


_RB = 1000


def _full(shape):
    return pl.BlockSpec(shape, lambda i: (0,) * len(shape))


def _rows(feat):
    return pl.BlockSpec((_RB, feat), lambda i: (i, 0))


def _deg_spec():
    return pl.BlockSpec((_NC, _RB, _L), lambda i: (0, i, 0))


def _part_spec(feat):
    return pl.BlockSpec((_NC, _RB, feat), lambda i: (0, i, 0))


def kernel(x, adj, W0_1, W1_1, b1, W0_2, W1_2, b2):
    n, f_in = x.shape
    e = adj.shape[1]
    h = W0_1.shape[1]
    c = W0_2.shape[1]
    grid = (n // _RB,)

    ept = e // _NW
    npad = _npad(n)
    Bd = _edge_batch(ept, _L, npad, 5)
    B1 = _edge_batch(ept, h, npad, 5)
    B2 = _edge_batch(ept, c, npad, 5)

    def _r3(v, b):
        return v.reshape(_NW, ept // b, b)

    degree = _make_degree(n, e)
    prop1 = _make_propagate(n, e, h)
    prop2 = _make_propagate(n, e, c)

    degm = degree(_r3(adj[0], Bd))

    tc1 = pl.pallas_call(
        _tc1_body,
        grid=grid,
        in_specs=[
            _rows(f_in), _full((f_in, h)), _full((f_in, h)), _full((1, h)),
            _deg_spec(),
        ],
        out_specs=[_rows(h), _rows(h)],
        out_shape=[
            jax.ShapeDtypeStruct((n, h), jnp.float32),
            jax.ShapeDtypeStruct((n, h), jnp.float32),
        ],
    )
    xw0b, z1 = tc1(x, W0_1, W1_1, b1.reshape(1, h), degm)

    p1 = prop1(z1, _r3(adj[0], B1), _r3(adj[1], B1))

    tc2 = pl.pallas_call(
        _tc2_body,
        grid=grid,
        in_specs=[
            _rows(h), _part_spec(h), _deg_spec(),
            _full((h, c)), _full((h, c)), _full((1, c)),
        ],
        out_specs=[_rows(c), _rows(c)],
        out_shape=[
            jax.ShapeDtypeStruct((n, c), jnp.float32),
            jax.ShapeDtypeStruct((n, c), jnp.float32),
        ],
    )
    hw0b, z2 = tc2(xw0b, p1, degm, W0_2, W1_2, b2.reshape(1, c))

    p2 = prop2(z2, _r3(adj[0], B2), _r3(adj[1], B2))

    tc3 = pl.pallas_call(
        _tc3_body,
        grid=grid,
        in_specs=[_rows(c), _part_spec(c), _deg_spec()],
        out_specs=_rows(c),
        out_shape=jax.ShapeDtypeStruct((n, c), jnp.float32),
    )
    return tc3(hw0b, p2, degm)
